# initial kernel scaffold (unmeasured)
import jax
import jax.numpy as jnp
from jax import lax
from jax.experimental import pallas as pl
from jax.experimental.pallas import tpu as pltpu

N_DEV = 4
B, Sq, Skv, Dh = 4, 256, 1024, 128
H = 8
D = 1024
SCALE = 0.08838834764831843
ROWS = B * Sq
CHUNK = ROWS // N_DEV


def kernel(x, Wq, Wo, K_ext, V_ext):
    idx = lax.axis_index("i")
    K_loc = lax.dynamic_slice_in_dim(K_ext, idx * H, H, axis=2)
    V_loc = lax.dynamic_slice_in_dim(V_ext, idx * H, H, axis=2)
    K_loc = jnp.transpose(K_loc, (0, 2, 1, 3))
    V_loc = jnp.transpose(V_loc, (0, 2, 1, 3))
    xm = x.reshape(ROWS, D)

    def body(x_ref, wq_ref, wo_ref, k_ref, v_ref, out_ref,
             attn_ref, comm_ref, send_sems, recv_sems):
        my = lax.axis_index("i")
        left = (my + N_DEV - 1) % N_DEV
        right = (my + 1) % N_DEV

        barrier_sem = pltpu.get_barrier_semaphore()
        for nbr in (left, right):
            pl.semaphore_signal(
                barrier_sem, inc=1,
                device_id=(nbr,), device_id_type=pl.DeviceIdType.MESH,
            )
        pl.semaphore_wait(barrier_sem, 2)

        q = jnp.dot(x_ref[...], wq_ref[...],
                    preferred_element_type=jnp.float32)
        for b in range(B):
            qb = q[b * Sq:(b + 1) * Sq, :]
            for h in range(H):
                qh = qb[:, h * Dh:(h + 1) * Dh]
                kh = k_ref[b, h]
                vh = v_ref[b, h]
                s = lax.dot_general(
                    qh, kh, (((1,), (1,)), ((), ())),
                    preferred_element_type=jnp.float32) * SCALE
                m = jnp.max(s, axis=-1, keepdims=True)
                p = jnp.exp(s - m)
                l = jnp.sum(p, axis=-1, keepdims=True)
                o = jnp.dot(p, vh, preferred_element_type=jnp.float32) / l
                attn_ref[b * Sq:(b + 1) * Sq, h * Dh:(h + 1) * Dh] = o
        out_ref[...] = jnp.dot(attn_ref[...], wo_ref[...],
                               preferred_element_type=jnp.float32)

        for t in range(N_DEV - 1):
            sc = (my + N_DEV - t) % N_DEV
            rc = (my + N_DEV - t - 1) % N_DEV
            rdma = pltpu.make_async_remote_copy(
                src_ref=out_ref.at[pl.ds(sc * CHUNK, CHUNK)],
                dst_ref=comm_ref.at[t],
                send_sem=send_sems.at[t],
                recv_sem=recv_sems.at[t],
                device_id=(right,),
                device_id_type=pl.DeviceIdType.MESH,
            )
            rdma.start()
            rdma.wait()
            cur = out_ref[pl.ds(rc * CHUNK, CHUNK), :]
            out_ref[pl.ds(rc * CHUNK, CHUNK), :] = cur + comm_ref[t]

        for t in range(N_DEV - 1):
            sc = (my + 1 + N_DEV - t) % N_DEV
            rdma = pltpu.make_async_remote_copy(
                src_ref=out_ref.at[pl.ds(sc * CHUNK, CHUNK)],
                dst_ref=out_ref.at[pl.ds(sc * CHUNK, CHUNK)],
                send_sem=send_sems.at[N_DEV - 1 + t],
                recv_sem=recv_sems.at[N_DEV - 1 + t],
                device_id=(right,),
                device_id_type=pl.DeviceIdType.MESH,
            )
            rdma.start()
            rdma.wait()

    out2 = pl.pallas_call(
        body,
        out_shape=jax.ShapeDtypeStruct((ROWS, D), jnp.float32),
        in_specs=[pl.BlockSpec(memory_space=pltpu.VMEM)] * 5,
        out_specs=pl.BlockSpec(memory_space=pltpu.VMEM),
        scratch_shapes=[
            pltpu.VMEM((ROWS, H * Dh), jnp.float32),
            pltpu.VMEM((N_DEV - 1, CHUNK, D), jnp.float32),
            pltpu.SemaphoreType.DMA((2 * (N_DEV - 1),)),
            pltpu.SemaphoreType.DMA((2 * (N_DEV - 1),)),
        ],
        compiler_params=pltpu.CompilerParams(collective_id=0),
    )(xm, Wq, Wo, K_loc, V_loc)
    return out2.reshape(B, Sq, D)


# baseline (device time: 154644 ns/iter reference)
import jax
import jax.numpy as jnp
from jax import lax
from jax.experimental import pallas as pl
from jax.experimental.pallas import tpu as pltpu

N_DEV = 4
B, Sq, Skv, Dh = 4, 256, 1024, 128
H = 8
D = 1024
SCALE = 0.08838834764831843
ROWS = B * Sq
CHUNK = ROWS // N_DEV


def kernel(x, Wq, Wo, K_ext, V_ext):
    idx = lax.axis_index("i")
    K_loc = lax.dynamic_slice_in_dim(K_ext, idx * H, H, axis=2)
    V_loc = lax.dynamic_slice_in_dim(V_ext, idx * H, H, axis=2)
    K_loc = jnp.transpose(K_loc, (0, 2, 1, 3))
    V_loc = jnp.transpose(V_loc, (0, 2, 1, 3))
    xm = x.reshape(ROWS, D)

    def body(x_ref, wq_ref, wo_ref, k_hbm, v_hbm, out_ref,
             attn_ref, kbuf, vbuf, comm_ref,
             copy_sems, send_sems, recv_sems):
        my = lax.axis_index("i")
        left = (my + N_DEV - 1) % N_DEV
        right = (my + 1) % N_DEV

        barrier_sem = pltpu.get_barrier_semaphore()
        for nbr in (left, right):
            pl.semaphore_signal(
                barrier_sem, inc=1,
                device_id=(nbr,), device_id_type=pl.DeviceIdType.MESH,
            )
        pl.semaphore_wait(barrier_sem, 2)

        def kv_copy(b, slot):
            ck = pltpu.make_async_copy(k_hbm.at[b], kbuf.at[slot],
                                       copy_sems.at[2 * slot])
            cv = pltpu.make_async_copy(v_hbm.at[b], vbuf.at[slot],
                                       copy_sems.at[2 * slot + 1])
            ck.start()
            cv.start()
            return ck, cv

        pending = kv_copy(0, 0)
        for b in range(B):
            slot = b % 2
            q_b = jnp.dot(x_ref[b * Sq:(b + 1) * Sq, :], wq_ref[...],
                          preferred_element_type=jnp.float32)
            pending[0].wait()
            pending[1].wait()
            if b + 1 < B:
                pending = kv_copy(b + 1, (b + 1) % 2)
            for h in range(H):
                qh = q_b[:, h * Dh:(h + 1) * Dh]
                kh = kbuf[slot, h]
                vh = vbuf[slot, h]
                s = lax.dot_general(
                    qh, kh, (((1,), (1,)), ((), ())),
                    preferred_element_type=jnp.float32) * SCALE
                m = jnp.max(s, axis=-1, keepdims=True)
                p = jnp.exp(s - m)
                l = jnp.sum(p, axis=-1, keepdims=True)
                o = jnp.dot(p, vh, preferred_element_type=jnp.float32) / l
                attn_ref[b * Sq:(b + 1) * Sq, h * Dh:(h + 1) * Dh] = o
        out_ref[...] = jnp.dot(attn_ref[...], wo_ref[...],
                               preferred_element_type=jnp.float32)

        for t in range(N_DEV - 1):
            sc = (my + N_DEV - t) % N_DEV
            rc = (my + N_DEV - t - 1) % N_DEV
            rdma = pltpu.make_async_remote_copy(
                src_ref=out_ref.at[pl.ds(sc * CHUNK, CHUNK)],
                dst_ref=comm_ref.at[t],
                send_sem=send_sems.at[t],
                recv_sem=recv_sems.at[t],
                device_id=(right,),
                device_id_type=pl.DeviceIdType.MESH,
            )
            rdma.start()
            rdma.wait()
            cur = out_ref[pl.ds(rc * CHUNK, CHUNK), :]
            out_ref[pl.ds(rc * CHUNK, CHUNK), :] = cur + comm_ref[t]

        for t in range(N_DEV - 1):
            sc = (my + 1 + N_DEV - t) % N_DEV
            rdma = pltpu.make_async_remote_copy(
                src_ref=out_ref.at[pl.ds(sc * CHUNK, CHUNK)],
                dst_ref=out_ref.at[pl.ds(sc * CHUNK, CHUNK)],
                send_sem=send_sems.at[N_DEV - 1 + t],
                recv_sem=recv_sems.at[N_DEV - 1 + t],
                device_id=(right,),
                device_id_type=pl.DeviceIdType.MESH,
            )
            rdma.start()
            rdma.wait()

    out2 = pl.pallas_call(
        body,
        out_shape=jax.ShapeDtypeStruct((ROWS, D), jnp.float32),
        in_specs=[
            pl.BlockSpec(memory_space=pltpu.VMEM),
            pl.BlockSpec(memory_space=pltpu.VMEM),
            pl.BlockSpec(memory_space=pltpu.VMEM),
            pl.BlockSpec(memory_space=pl.ANY),
            pl.BlockSpec(memory_space=pl.ANY),
        ],
        out_specs=pl.BlockSpec(memory_space=pltpu.VMEM),
        scratch_shapes=[
            pltpu.VMEM((ROWS, H * Dh), jnp.float32),
            pltpu.VMEM((2, H, Skv, Dh), jnp.float32),
            pltpu.VMEM((2, H, Skv, Dh), jnp.float32),
            pltpu.VMEM((N_DEV - 1, CHUNK, D), jnp.float32),
            pltpu.SemaphoreType.DMA((4,)),
            pltpu.SemaphoreType.DMA((2 * (N_DEV - 1),)),
            pltpu.SemaphoreType.DMA((2 * (N_DEV - 1),)),
        ],
        compiler_params=pltpu.CompilerParams(collective_id=0),
    )(xm, Wq, Wo, K_loc, V_loc)
    return out2.reshape(B, Sq, D)


# device time: 90408 ns/iter; 1.7105x vs baseline; 1.7105x over previous
import jax
import jax.numpy as jnp
from jax import lax
from jax.experimental import pallas as pl
from jax.experimental.pallas import tpu as pltpu

N_DEV = 4
B, Sq, Skv, Dh = 4, 256, 1024, 128
H = 8
HG = 32
D = 1024
SCALE = 0.08838834764831843
ROWS = B * Sq
CHUNK = ROWS // N_DEV


def kernel(x, Wq, Wo, K_ext, V_ext):
    xm = x.reshape(ROWS, D)

    def body(x_ref, wq_ref, wo_ref, k_hbm, v_hbm, out_ref,
             kbuf, vbuf, comm_ref,
             copy_sems, rs_send_sems, rs_recv_sems, bc_send_sems,
             bc_recv_sems):
        my = lax.axis_index("i")
        left = (my + N_DEV - 1) % N_DEV
        right = (my + 1) % N_DEV
        diag = (my + 2) % N_DEV
        hs = my * H

        barrier_sem = pltpu.get_barrier_semaphore()
        for nbr in (left, right):
            pl.semaphore_signal(
                barrier_sem, inc=1,
                device_id=(nbr,), device_id_type=pl.DeviceIdType.MESH,
            )
        pl.semaphore_wait(barrier_sem, 2)

        def kv_copy(b, slot):
            ck = pltpu.make_async_copy(
                k_hbm.at[b, :, pl.ds(hs, H), :], kbuf.at[slot],
                copy_sems.at[2 * slot])
            cv = pltpu.make_async_copy(
                v_hbm.at[b, :, pl.ds(hs, H), :], vbuf.at[slot],
                copy_sems.at[2 * slot + 1])
            ck.start()
            cv.start()
            return ck, cv

        def batch_index(j):
            if j < N_DEV - 1:
                return (my + N_DEV - j) % N_DEV
            return (my + 1) % N_DEV

        rdmas = []
        pending = kv_copy(batch_index(0), 0)
        for j in range(N_DEV):
            b = batch_index(j)
            slot = j % 2
            q_b = jnp.dot(x_ref[pl.ds(b * Sq, Sq), :], wq_ref[...],
                          preferred_element_type=jnp.float32)
            pending[0].wait()
            pending[1].wait()
            if j + 1 < N_DEV:
                pending = kv_copy(batch_index(j + 1), (j + 1) % 2)
            heads = []
            for h in range(H):
                qh = q_b[:, h * Dh:(h + 1) * Dh]
                kh = kbuf[slot, :, h, :]
                vh = vbuf[slot, :, h, :]
                s = lax.dot_general(
                    qh, kh, (((1,), (1,)), ((), ())),
                    preferred_element_type=jnp.float32) * SCALE
                m = jnp.max(s, axis=-1, keepdims=True)
                p = jnp.exp(s - m)
                l = jnp.sum(p, axis=-1, keepdims=True)
                heads.append(
                    jnp.dot(p, vh, preferred_element_type=jnp.float32) / l)
            attn_b = jnp.concatenate(heads, axis=1)
            part = jnp.dot(attn_b, wo_ref[...],
                           preferred_element_type=jnp.float32)

            if j == 0:
                out_ref[pl.ds(b * CHUNK, CHUNK), :] = part
            else:
                prev = rdmas[j - 1]
                prev.wait_recv()
                out_ref[pl.ds(b * CHUNK, CHUNK), :] = (
                    part + comm_ref[j - 1])
            if j < N_DEV - 1:
                rdma = pltpu.make_async_remote_copy(
                    src_ref=out_ref.at[pl.ds(b * CHUNK, CHUNK)],
                    dst_ref=comm_ref.at[j],
                    send_sem=rs_send_sems.at[j],
                    recv_sem=rs_recv_sems.at[j],
                    device_id=(right,),
                    device_id_type=pl.DeviceIdType.MESH,
                )
                rdma.start()
                rdmas.append(rdma)

        own = (my + 1) % N_DEV
        own_slice = pl.ds(own * CHUNK, CHUNK)
        for k, tgt in enumerate((right, left, diag)):
            bc = pltpu.make_async_remote_copy(
                src_ref=out_ref.at[own_slice],
                dst_ref=out_ref.at[own_slice],
                send_sem=bc_send_sems.at[k],
                recv_sem=bc_recv_sems.at[k],
                device_id=(tgt,),
                device_id_type=pl.DeviceIdType.MESH,
            )
            bc.start()
            rdmas.append(bc)
        for k in range(3):
            rdmas[N_DEV - 1 + k].wait_recv()
        for r in rdmas:
            r.wait_send()

    out2 = pl.pallas_call(
        body,
        out_shape=jax.ShapeDtypeStruct((ROWS, D), jnp.float32),
        in_specs=[
            pl.BlockSpec(memory_space=pltpu.VMEM),
            pl.BlockSpec(memory_space=pltpu.VMEM),
            pl.BlockSpec(memory_space=pltpu.VMEM),
            pl.BlockSpec(memory_space=pl.ANY),
            pl.BlockSpec(memory_space=pl.ANY),
        ],
        out_specs=pl.BlockSpec(memory_space=pltpu.VMEM),
        scratch_shapes=[
            pltpu.VMEM((2, Skv, H, Dh), jnp.float32),
            pltpu.VMEM((2, Skv, H, Dh), jnp.float32),
            pltpu.VMEM((N_DEV - 1, CHUNK, D), jnp.float32),
            pltpu.SemaphoreType.DMA((4,)),
            pltpu.SemaphoreType.DMA((N_DEV - 1,)),
            pltpu.SemaphoreType.DMA((N_DEV - 1,)),
            pltpu.SemaphoreType.DMA((3,)),
            pltpu.SemaphoreType.DMA((3,)),
        ],
        compiler_params=pltpu.CompilerParams(collective_id=0),
    )(xm, Wq, Wo, K_ext, V_ext)
    return out2.reshape(B, Sq, D)
